# baseline (speedup 1.0000x reference)
import jax
import jax.numpy as jnp
from jax.experimental import pallas as pl
from jax.experimental.pallas import tpu as pltpu

_B, _C, _H, _W = 16, 512, 32, 32
_D = 256


def _pos_kernel(col_ref, row_ref, out_ref):
    col = col_ref[0:_W, :]
    row = row_ref[0:_H, :]
    out_ref[0, :, :, 0:_D] = jnp.broadcast_to(col[None, :, :], (_H, _W, _D))
    out_ref[0, :, :, _D:_C] = jnp.broadcast_to(row[:, None, :], (_H, _W, _D))


def kernel(x, row_embed, col_embed):
    b = x.shape[0]
    out = pl.pallas_call(
        _pos_kernel,
        grid=(b,),
        in_specs=[
            pl.BlockSpec(col_embed.shape, lambda i: (0, 0)),
            pl.BlockSpec(row_embed.shape, lambda i: (0, 0)),
        ],
        out_specs=pl.BlockSpec((1, _H, _W, _C), lambda i: (i, 0, 0, 0)),
        out_shape=jax.ShapeDtypeStruct((b, _H, _W, _C), jnp.float32),
    )(col_embed, row_embed)
    return jnp.transpose(out, (0, 3, 1, 2))


# speedup vs baseline: 1.1297x; 1.1297x over previous
import jax
import jax.numpy as jnp
from jax.experimental import pallas as pl
from jax.experimental.pallas import tpu as pltpu

_B, _C, _H, _W = 16, 512, 32, 32
_D = 256


def _pos_kernel(col_ref, row_ref, out_hbm, scratch, sem):
    col = col_ref[0:_W, :]
    row = row_ref[0:_H, :]
    scratch[:, :, 0:_D] = jnp.broadcast_to(col[None, :, :], (_H, _W, _D))
    scratch[:, :, _D:_C] = jnp.broadcast_to(row[:, None, :], (_H, _W, _D))
    for b in range(_B):
        pltpu.make_async_copy(scratch, out_hbm.at[b], sem.at[b]).start()
    for b in range(_B):
        pltpu.make_async_copy(scratch, out_hbm.at[b], sem.at[b]).wait()


def kernel(x, row_embed, col_embed):
    b = x.shape[0]
    out = pl.pallas_call(
        _pos_kernel,
        in_specs=[
            pl.BlockSpec(memory_space=pltpu.VMEM),
            pl.BlockSpec(memory_space=pltpu.VMEM),
        ],
        out_specs=pl.BlockSpec(memory_space=pl.ANY),
        out_shape=jax.ShapeDtypeStruct((b, _H, _W, _C), jnp.float32),
        scratch_shapes=[
            pltpu.VMEM((_H, _W, _C), jnp.float32),
            pltpu.SemaphoreType.DMA((_B,)),
        ],
    )(col_embed, row_embed)
    return jnp.transpose(out, (0, 3, 1, 2))
